# baseline (speedup 1.0000x reference)
import jax
import jax.numpy as jnp
from jax.experimental import pallas as pl

N = 65536
C = 128
R = N // C
RM = 2 * R


def _row_iota(rows):
    return jax.lax.broadcasted_iota(jnp.int32, (rows, 1), 0)


def _lane_iota():
    return jax.lax.broadcasted_iota(jnp.int32, (1, C), 1)


def _bitmask(rows, b):
    if b >= C:
        return (_row_iota(rows) & (b // C)) == 0
    return (_lane_iota() & b) == 0


def _partner(a, d, low):
    if d >= C:
        dr = d // C
        return jnp.where(low, jnp.roll(a, -dr, axis=0), jnp.roll(a, dr, axis=0))
    return jnp.where(low, jnp.roll(a, -d, axis=1), jnp.roll(a, d, axis=1))


def _low_asc(shape, k, d, descending):
    rows = shape[0]
    low = _bitmask(rows, d)
    asc = _bitmask(rows, k)
    if descending:
        asc = jnp.logical_not(asc)
    return low, asc


def _cmpx(key, val, k, d, descending):
    low, asc = _low_asc(key.shape, k, d, descending)
    pk = _partner(key, d, low)
    keepmin = low == asc
    mn = jnp.minimum(key, pk)
    mx = jnp.maximum(key, pk)
    newkey = jnp.where(keepmin, mn, mx)
    if val is None:
        return newkey, None
    pv = _partner(val, d, low)
    alt = (key < pk) | ((key == pk) & low)
    newval = jnp.where(keepmin == alt, val, pv)
    return newkey, newval


def _bitonic_sort(key, val=None, descending=False):
    n = key.shape[0] * key.shape[1]
    k = 2
    while k <= n:
        d = k // 2
        while d >= 1:
            key, val = _cmpx(key, val, k, d, descending)
            d //= 2
        k *= 2
    return key, val


def _shift_right(a, d, zero):
    rows = a.shape[0]
    if d >= C:
        dr = d // C
        b = jnp.roll(a, dr, axis=0)
        return jnp.where(_row_iota(rows) >= dr, b, zero)
    b = jnp.roll(a, d, axis=1)
    brow = jnp.roll(b, 1, axis=0)
    return jnp.where(
        _lane_iota() >= d, b, jnp.where(_row_iota(rows) >= 1, brow, zero)
    )


def _shift_left(a, d, fill):
    rows = a.shape[0]
    if d >= C:
        dr = d // C
        b = jnp.roll(a, -dr, axis=0)
        return jnp.where(_row_iota(rows) < rows - dr, b, fill)
    b = jnp.roll(a, -d, axis=1)
    brow = jnp.roll(b, -1, axis=0)
    return jnp.where(
        _lane_iota() < C - d, b, jnp.where(_row_iota(rows) < rows - 1, brow, fill)
    )


def _wasserstein_kernel(x_ref, y_ref, w_ref, out_ref):
    x = x_ref[...]
    y = y_ref[...]
    w = w_ref[...]

    us, ws = _bitonic_sort(x, w)
    vdesc, _ = _bitonic_sort(y, descending=True)

    cum = ws
    d = 1
    while d < N:
        cum = cum + _shift_right(cum, d, 0.0)
        d *= 2
    total = jax.lax.slice(cum, (R - 1, C - 1), (R, C))
    ucum = cum / total

    flat = _row_iota(R) * C + _lane_iota()
    vkey_desc = (N - flat).astype(jnp.float32) * (1.0 / N)
    key = jnp.concatenate([ucum, vkey_desc], axis=0)
    val = jnp.concatenate([us, vdesc], axis=0)
    tag = jnp.concatenate(
        [jnp.zeros((R, C), jnp.int32), jnp.ones((R, C), jnp.int32)], axis=0
    )
    d = N
    while d >= 1:
        low, _ = _low_asc(key.shape, 2 * N, d, False)
        pk = _partner(key, d, low)
        pv = _partner(val, d, low)
        pt = _partner(tag, d, low)
        keepmin = low
        alt = (key < pk) | ((key == pk) & low)
        take_own = keepmin == alt
        key = jnp.where(keepmin, jnp.minimum(key, pk), jnp.maximum(key, pk))
        val = jnp.where(take_own, val, pv)
        tag = jnp.where(take_own, tag, pt)
        d //= 2

    us_last = jax.lax.slice(us, (R - 1, C - 1), (R, C))
    vs_last = jax.lax.slice(vdesc, (0, 0), (1, 1))
    uq = jnp.where(tag == 0, val, 0.0)
    uh = jnp.where(tag == 0, 1, 0)
    vq = jnp.where(tag == 1, val, 0.0)
    vh = jnp.where(tag == 1, 1, 0)
    ufill = us_last
    vfill = vs_last
    d = 1
    while d < 2 * N:
        uq_s = _shift_left(uq, d, ufill)
        uh_s = _shift_left(uh, d, 1)
        vq_s = _shift_left(vq, d, vfill)
        vh_s = _shift_left(vh, d, 1)
        uq = jnp.where(uh == 1, uq, uq_s)
        uh = uh | uh_s
        vq = jnp.where(vh == 1, vq, vq_s)
        vh = vh | vh_s
        d *= 2

    delta = key - _shift_right(key, 1, 0.0)
    diff = uq - vq
    out_ref[...] = jnp.sum(delta * diff * diff, keepdims=True)


def kernel(x, y, x_weights):
    x2 = x.reshape(R, C)
    y2 = y.reshape(R, C)
    w2 = x_weights.reshape(R, C)
    out = pl.pallas_call(
        _wasserstein_kernel,
        out_shape=jax.ShapeDtypeStruct((1, 1), jnp.float32),
    )(x2, y2, w2)
    return out[0, 0]


# speedup vs baseline: 1.0454x; 1.0454x over previous
import jax
import jax.numpy as jnp
from jax.experimental import pallas as pl

N = 65536
C = 128
R = N // C
RM = 2 * R


def _row_iota(rows):
    return jax.lax.broadcasted_iota(jnp.int32, (rows, 1), 0)


def _lane_iota():
    return jax.lax.broadcasted_iota(jnp.int32, (1, C), 1)


def _bitmask(rows, b):
    if b >= C:
        return (_row_iota(rows) & (b // C)) == 0
    return (_lane_iota() & b) == 0


def _partner(a, d, low):
    if d >= C:
        dr = d // C
        return jnp.where(low, jnp.roll(a, -dr, axis=0), jnp.roll(a, dr, axis=0))
    return jnp.where(low, jnp.roll(a, -d, axis=1), jnp.roll(a, d, axis=1))


def _low_asc(shape, k, d, descending):
    rows = shape[0]
    low = _bitmask(rows, d)
    asc = _bitmask(rows, k)
    if descending:
        asc = jnp.logical_not(asc)
    return low, asc


def _cmpx(key, val, k, d, descending):
    low, asc = _low_asc(key.shape, k, d, descending)
    pk = _partner(key, d, low)
    keepmin = low == asc
    mn = jnp.minimum(key, pk)
    mx = jnp.maximum(key, pk)
    newkey = jnp.where(keepmin, mn, mx)
    if val is None:
        return newkey, None
    pv = _partner(val, d, low)
    alt = (key < pk) | ((key == pk) & low)
    newval = jnp.where(keepmin == alt, val, pv)
    return newkey, newval


def _bitonic_sort(key, val=None, descending=False):
    n = key.shape[0] * key.shape[1]
    k = 2
    while k <= n:
        d = k // 2
        while d >= 1:
            key, val = _cmpx(key, val, k, d, descending)
            d //= 2
        k *= 2
    return key, val


def _shift_right(a, d, zero):
    rows = a.shape[0]
    if d >= C:
        dr = d // C
        b = jnp.roll(a, dr, axis=0)
        return jnp.where(_row_iota(rows) >= dr, b, zero)
    b = jnp.roll(a, d, axis=1)
    brow = jnp.roll(b, 1, axis=0)
    return jnp.where(
        _lane_iota() >= d, b, jnp.where(_row_iota(rows) >= 1, brow, zero)
    )


def _shift_left(a, d, fill):
    rows = a.shape[0]
    if d >= C:
        dr = d // C
        b = jnp.roll(a, -dr, axis=0)
        return jnp.where(_row_iota(rows) < rows - dr, b, fill)
    b = jnp.roll(a, -d, axis=1)
    brow = jnp.roll(b, -1, axis=0)
    return jnp.where(
        _lane_iota() < C - d, b, jnp.where(_row_iota(rows) < rows - 1, brow, fill)
    )


def _wasserstein_kernel(x_ref, y_ref, w_ref, out_ref):
    x = x_ref[...]
    y = y_ref[...]
    w = w_ref[...]

    us, ws = _bitonic_sort(x, w)
    vdesc, _ = _bitonic_sort(y, descending=True)

    cum = ws
    d = 1
    while d < N:
        cum = cum + _shift_right(cum, d, 0.0)
        d *= 2
    total = jax.lax.slice(cum, (R - 1, C - 1), (R, C))
    ucum = cum / total

    flat = _row_iota(R) * C + _lane_iota()
    vkey_desc = (N - flat).astype(jnp.float32) * (1.0 / N)
    key = jnp.concatenate([ucum, vkey_desc], axis=0)
    val = jnp.concatenate([us, vdesc], axis=0)
    tag = jnp.concatenate(
        [jnp.zeros((R, C), jnp.int32), jnp.ones((R, C), jnp.int32)], axis=0
    )
    d = N
    while d >= 1:
        low, _ = _low_asc(key.shape, 2 * N, d, False)
        pk = _partner(key, d, low)
        pv = _partner(val, d, low)
        pt = _partner(tag, d, low)
        keepmin = low
        alt = (key < pk) | ((key == pk) & low)
        take_own = keepmin == alt
        key = jnp.where(keepmin, jnp.minimum(key, pk), jnp.maximum(key, pk))
        val = jnp.where(take_own, val, pv)
        tag = jnp.where(take_own, tag, pt)
        d //= 2

    us_last = jax.lax.slice(us, (R - 1, C - 1), (R, C))
    vs_last = jax.lax.slice(vdesc, (0, 0), (1, 1))
    uq = jnp.where(tag == 0, val, us_last)
    vq = jnp.where(tag == 1, val, vs_last)
    d = 1
    while d < 2 * N:
        uq = jnp.minimum(uq, _shift_left(uq, d, us_last))
        vq = jnp.minimum(vq, _shift_left(vq, d, vs_last))
        d *= 2

    delta = key - _shift_right(key, 1, 0.0)
    diff = uq - vq
    out_ref[...] = jnp.sum(delta * diff * diff, keepdims=True)


def kernel(x, y, x_weights):
    x2 = x.reshape(R, C)
    y2 = y.reshape(R, C)
    w2 = x_weights.reshape(R, C)
    out = pl.pallas_call(
        _wasserstein_kernel,
        out_shape=jax.ShapeDtypeStruct((1, 1), jnp.float32),
    )(x2, y2, w2)
    return out[0, 0]


# speedup vs baseline: 1.0672x; 1.0208x over previous
import jax
import jax.numpy as jnp
from jax.experimental import pallas as pl

N = 65536
C = 128
R = N // C
RM = 2 * R


def _row_iota(rows):
    return jax.lax.broadcasted_iota(jnp.int32, (rows, 1), 0)


def _lane_iota():
    return jax.lax.broadcasted_iota(jnp.int32, (1, C), 1)


def _bitmask(rows, b):
    if b >= C:
        return (_row_iota(rows) & (b // C)) == 0
    return (_lane_iota() & b) == 0


def _partner(a, d, low):
    if d >= C:
        dr = d // C
        return jnp.where(low, jnp.roll(a, -dr, axis=0), jnp.roll(a, dr, axis=0))
    return jnp.where(low, jnp.roll(a, -d, axis=1), jnp.roll(a, d, axis=1))


def _low_asc(shape, k, d, descending):
    rows = shape[0]
    low = _bitmask(rows, d)
    asc = _bitmask(rows, k)
    if descending:
        asc = jnp.logical_not(asc)
    return low, asc


def _cmpx(key, val, k, d, descending):
    low, asc = _low_asc(key.shape, k, d, descending)
    pk = _partner(key, d, low)
    keepmin = low == asc
    mn = jnp.minimum(key, pk)
    mx = jnp.maximum(key, pk)
    newkey = jnp.where(keepmin, mn, mx)
    if val is None:
        return newkey, None
    pv = _partner(val, d, low)
    alt = (key < pk) | ((key == pk) & low)
    newval = jnp.where(keepmin == alt, val, pv)
    return newkey, newval


def _bitonic_sort(key, val=None, descending=False):
    n = key.shape[0] * key.shape[1]
    k = 2
    while k <= n:
        d = k // 2
        while d >= 1:
            key, val = _cmpx(key, val, k, d, descending)
            d //= 2
        k *= 2
    return key, val


def _shift_right(a, d, zero):
    rows = a.shape[0]
    if d >= C:
        dr = d // C
        b = jnp.roll(a, dr, axis=0)
        return jnp.where(_row_iota(rows) >= dr, b, zero)
    b = jnp.roll(a, d, axis=1)
    brow = jnp.roll(b, 1, axis=0)
    return jnp.where(
        _lane_iota() >= d, b, jnp.where(_row_iota(rows) >= 1, brow, zero)
    )


def _shift_left(a, d, fill):
    rows = a.shape[0]
    if d >= C:
        dr = d // C
        b = jnp.roll(a, -dr, axis=0)
        return jnp.where(_row_iota(rows) < rows - dr, b, fill)
    b = jnp.roll(a, -d, axis=1)
    brow = jnp.roll(b, -1, axis=0)
    return jnp.where(
        _lane_iota() < C - d, b, jnp.where(_row_iota(rows) < rows - 1, brow, fill)
    )


def _wasserstein_kernel(x_ref, y_ref, w_ref, out_ref):
    x = x_ref[...]
    y = y_ref[...]
    w = w_ref[...]

    us, ws = x, w
    vdesc = y
    k = 2
    while k <= N:
        d = k // 2
        while d >= 1:
            us, ws = _cmpx(us, ws, k, d, False)
            vdesc, _ = _cmpx(vdesc, None, k, d, True)
            d //= 2
        k *= 2

    cum = ws
    d = 1
    while d < N:
        cum = cum + _shift_right(cum, d, 0.0)
        d *= 2
    total = jax.lax.slice(cum, (R - 1, C - 1), (R, C))
    ucum = cum / total

    flat = _row_iota(R) * C + _lane_iota()
    vkey_desc = (N - flat).astype(jnp.float32) * (1.0 / N)
    key = jnp.concatenate([ucum, vkey_desc], axis=0)
    val = jnp.concatenate([us, vdesc], axis=0)
    tag = jnp.concatenate(
        [jnp.zeros((R, C), jnp.int32), jnp.ones((R, C), jnp.int32)], axis=0
    )
    d = N
    while d >= 1:
        low, _ = _low_asc(key.shape, 2 * N, d, False)
        pk = _partner(key, d, low)
        pv = _partner(val, d, low)
        pt = _partner(tag, d, low)
        keepmin = low
        alt = (key < pk) | ((key == pk) & low)
        take_own = keepmin == alt
        key = jnp.where(keepmin, jnp.minimum(key, pk), jnp.maximum(key, pk))
        val = jnp.where(take_own, val, pv)
        tag = jnp.where(take_own, tag, pt)
        d //= 2

    us_last = jax.lax.slice(us, (R - 1, C - 1), (R, C))
    vs_last = jax.lax.slice(vdesc, (0, 0), (1, 1))
    uq = jnp.where(tag == 0, val, us_last)
    vq = jnp.where(tag == 1, val, vs_last)
    d = 1
    while d < 2 * N:
        uq = jnp.minimum(uq, _shift_left(uq, d, us_last))
        vq = jnp.minimum(vq, _shift_left(vq, d, vs_last))
        d *= 2

    delta = key - _shift_right(key, 1, 0.0)
    diff = uq - vq
    out_ref[...] = jnp.sum(delta * diff * diff, keepdims=True)


def kernel(x, y, x_weights):
    x2 = x.reshape(R, C)
    y2 = y.reshape(R, C)
    w2 = x_weights.reshape(R, C)
    out = pl.pallas_call(
        _wasserstein_kernel,
        out_shape=jax.ShapeDtypeStruct((1, 1), jnp.float32),
    )(x2, y2, w2)
    return out[0, 0]
